# Initial kernel scaffold; baseline (speedup 1.0000x reference)
import functools

import jax
import jax.numpy as jnp
from jax import lax
from jax.experimental import pallas as pl
from jax.experimental.pallas import tpu as pltpu
from jax.experimental.pallas import tpu_sc as plsc

N = 10000
D = 128
H = 64
T = 28
Q = 3
K = 50

NC = 2
NS = 16
L = 16
NW = NC * NS

NP = 10240
DP = 144
F = 96
TP = 32

EB = 128
NB = 80
BLK = 512


def _edge_body(src_hbm, dst_hbm, xp_hbm, agg_hbm,
               sidx_v, didx_v, rows_v, agg_sh, sem, *, epw):
    cid = lax.axis_index("c")
    sid = lax.axis_index("s")
    wid = sid * NC + cid

    def zrow(i, _):
        for c in range(DP // L):
            rows_v[i, pl.ds(c * L, L)] = jnp.zeros((L,), jnp.float32)
        return 0
    lax.fori_loop(0, EB, zrow, 0)
    zrows = NP // NS
    for z in range(zrows // EB):
        pltpu.sync_copy(rows_v, agg_sh.at[pl.ds(sid * zrows + z * EB, EB)])
    plsc.subcore_barrier()

    e0 = wid * epw

    def body(j, _):
        base = e0 + j * EB
        pltpu.sync_copy(src_hbm.at[pl.ds(base, EB)], sidx_v)
        pltpu.sync_copy(dst_hbm.at[pl.ds(base, EB)], didx_v)
        pltpu.async_copy(xp_hbm.at[sidx_v], rows_v, sem).wait()
        pltpu.sync_copy(rows_v, agg_sh.at[didx_v], add=True)
        return 0
    lax.fori_loop(0, epw // EB, body, 0)
    plsc.subcore_barrier()

    pltpu.sync_copy(agg_sh.at[pl.ds(sid * zrows, zrows)],
                    agg_hbm.at[cid, pl.ds(sid * zrows, zrows)])


def _dense_body(xp_ref, agg_ref, sc_ref, msk_ref, valid_ref,
                ws_ref, wn_ref, wp_ref, b_ref, r_ref,
                wt_ref, base_ref, w_ref):
    a = agg_ref[0] + agg_ref[1]
    deg = jnp.maximum(a[:, D:D + 1], 1.0)
    agg = a[:, :D] / deg
    xb = xp_ref[...][:, :D]
    h = jnp.maximum(xb @ ws_ref[...] + agg @ wn_ref[...], 0.0)
    out96 = h @ wp_ref[...] + b_ref[...]
    sc = sc_ref[...]
    scb = sc @ r_ref[...]
    inv = (1.0 / sc) @ r_ref[...]
    m = msk_ref[...] > 0.0
    wt_ref[...] = out96 * scb * valid_ref[...]
    base_ref[...] = jnp.where(m, 0.0, out96)
    w_ref[...] = jnp.where(m, inv, 0.0)


def _kb_body(kbt_hbm, wt_hbm, base_hbm, w_hbm, out_hbm,
             kb_v, acc_v, bb_v, ww_v, sem, sem2):
    cid = lax.axis_index("c")
    sid = lax.axis_index("s")
    wid = sid * NC + cid
    npw = NP // NW
    n0 = wid * npw

    def batch(j, _):
        nb = n0 + j * NB
        bidx = nb // NB
        pltpu.sync_copy(kbt_hbm.at[bidx], kb_v)
        cb = pltpu.async_copy(base_hbm.at[pl.ds(nb, NB)], bb_v, sem2)
        cw = pltpu.async_copy(w_hbm.at[pl.ds(nb, NB)], ww_v, sem2)
        pltpu.async_copy(wt_hbm.at[kb_v.at[0]], acc_v, sem).wait()

        def kbody(k, _):
            pltpu.async_copy(wt_hbm.at[kb_v.at[k]], acc_v, sem,
                             add=True).wait()
            return 0
        lax.fori_loop(1, K, kbody, 0)
        cb.wait()
        cw.wait()

        def comb(i, _):
            for c in range(F // L):
                s = pl.ds(c * L, L)
                acc_v[i, s] = bb_v[i, s] + acc_v[i, s] * ww_v[i, s]
            return 0
        lax.fori_loop(0, NB, comb, 0)
        pltpu.sync_copy(acc_v, out_hbm.at[pl.ds(nb, NB)])
        return 0
    lax.fori_loop(0, npw // NB, batch, 0)


def kernel(x, edge_index, keybom, scaler, key_aggregation_status,
           W_self, W_neigh, W_proj, b_proj):
    f32 = jnp.float32
    i32 = jnp.int32
    E = edge_index.shape[1]
    epw = -(-E // (NW * EB)) * EB
    EP = epw * NW

    xp = jnp.zeros((NP, DP), f32).at[:N, :D].set(x).at[:N, D].set(1.0)
    srcp = jnp.full((EP,), N, i32).at[:E].set(edge_index[0])
    dstp = jnp.full((EP,), N, i32).at[:E].set(edge_index[1])
    kb = jnp.where(keybom < 0, N, keybom)
    kbt3 = (jnp.full((K, NP), N, i32).at[:, :N].set(kb.T)
            .reshape(K, NP // NB, NB).transpose(1, 0, 2))
    scp = jnp.ones((NP, TP), f32).at[:N, :T].set(scaler)
    mskf = jnp.zeros((NP, 1), f32).at[:N].set(
        (key_aggregation_status > 0).astype(f32))
    validf = jnp.zeros((NP, 1), f32).at[:N, :].set(1.0)
    wp96 = jnp.zeros((H, F), f32).at[:, :T * Q].set(W_proj)
    b96 = jnp.zeros((1, F), f32).at[0, :T * Q].set(b_proj)
    rmat = (jnp.arange(F)[None, :] // Q == jnp.arange(TP)[:, None]).astype(f32)

    mesh = plsc.VectorSubcoreMesh(core_axis_name="c", subcore_axis_name="s",
                                  num_cores=NC, num_subcores=NS)

    edge_fn = pl.kernel(
        functools.partial(_edge_body, epw=epw),
        out_type=jax.ShapeDtypeStruct((NC, NP, DP), f32),
        mesh=mesh,
        scratch_types=[
            pltpu.VMEM((EB,), i32),
            pltpu.VMEM((EB,), i32),
            pltpu.VMEM((EB, DP), f32),
            pltpu.VMEM_SHARED((NP, DP), f32),
            pltpu.SemaphoreType.DMA,
        ],
    )
    agg2 = edge_fn(srcp, dstp, xp)

    grid = NP // BLK
    wt, base, w = pl.pallas_call(
        _dense_body,
        grid=(grid,),
        in_specs=[
            pl.BlockSpec((BLK, DP), lambda i: (i, 0)),
            pl.BlockSpec((NC, BLK, DP), lambda i: (0, i, 0)),
            pl.BlockSpec((BLK, TP), lambda i: (i, 0)),
            pl.BlockSpec((BLK, 1), lambda i: (i, 0)),
            pl.BlockSpec((BLK, 1), lambda i: (i, 0)),
            pl.BlockSpec((D, H), lambda i: (0, 0)),
            pl.BlockSpec((D, H), lambda i: (0, 0)),
            pl.BlockSpec((H, F), lambda i: (0, 0)),
            pl.BlockSpec((1, F), lambda i: (0, 0)),
            pl.BlockSpec((TP, F), lambda i: (0, 0)),
        ],
        out_specs=[
            pl.BlockSpec((BLK, F), lambda i: (i, 0)),
            pl.BlockSpec((BLK, F), lambda i: (i, 0)),
            pl.BlockSpec((BLK, F), lambda i: (i, 0)),
        ],
        out_shape=[
            jax.ShapeDtypeStruct((NP, F), f32),
            jax.ShapeDtypeStruct((NP, F), f32),
            jax.ShapeDtypeStruct((NP, F), f32),
        ],
    )(xp, agg2, scp, mskf, validf, W_self, W_neigh, wp96, b96, rmat)

    kb_fn = pl.kernel(
        _kb_body,
        out_type=jax.ShapeDtypeStruct((NP, F), f32),
        mesh=mesh,
        scratch_types=[
            pltpu.VMEM((K, NB), i32),
            pltpu.VMEM((NB, F), f32),
            pltpu.VMEM((NB, F), f32),
            pltpu.VMEM((NB, F), f32),
            pltpu.SemaphoreType.DMA,
            pltpu.SemaphoreType.DMA,
        ],
    )
    outp = kb_fn(kbt3, wt, base, w)

    return outp[:N, :T * Q].reshape(N, T, Q)


# speedup vs baseline: 3.6740x; 3.6740x over previous
import functools

import jax
import jax.numpy as jnp
from jax import lax
from jax.experimental import pallas as pl
from jax.experimental.pallas import tpu as pltpu
from jax.experimental.pallas import tpu_sc as plsc

N = 10000
D = 128
H = 64
T = 28
Q = 3
K = 50

NC = 2
NS = 16
L = 16
NW = NC * NS

NP = 10240
DP = 144
F = 96
TP = 32

EB = 128
NB = 80
BLK = 512


def _edge_body(src_hbm, dst_hbm, xp_hbm, agg_hbm,
               sidx_v, didx_v, rows_v, agg_sh, sem, *, epw):
    cid = lax.axis_index("c")
    sid = lax.axis_index("s")
    wid = sid * NC + cid

    def zrow(i, _):
        for c in range(DP // L):
            rows_v[i, pl.ds(c * L, L)] = jnp.zeros((L,), jnp.float32)
        return 0
    lax.fori_loop(0, EB, zrow, 0)
    zrows = NP // NS
    for z in range(zrows // EB):
        pltpu.sync_copy(rows_v, agg_sh.at[pl.ds(sid * zrows + z * EB, EB)])
    plsc.subcore_barrier()

    e0 = wid * epw

    def body(j, _):
        base = e0 + j * EB
        pltpu.sync_copy(src_hbm.at[pl.ds(base, EB)], sidx_v)
        pltpu.sync_copy(dst_hbm.at[pl.ds(base, EB)], didx_v)
        pltpu.async_copy(xp_hbm.at[sidx_v], rows_v, sem).wait()
        pltpu.sync_copy(rows_v, agg_sh.at[didx_v], add=True)
        return 0
    lax.fori_loop(0, epw // EB, body, 0)
    plsc.subcore_barrier()

    pltpu.sync_copy(agg_sh.at[pl.ds(sid * zrows, zrows)],
                    agg_hbm.at[cid, pl.ds(sid * zrows, zrows)])


def _dense_body(xp_ref, agg_ref, sc_ref, msk_ref, valid_ref,
                ws_ref, wn_ref, wp_ref, b_ref, r_ref,
                wt_ref, base_ref, w_ref):
    a = agg_ref[0] + agg_ref[1]
    deg = jnp.maximum(a[:, D:D + 1], 1.0)
    agg = a[:, :D] / deg
    xb = xp_ref[...][:, :D]
    h = jnp.maximum(xb @ ws_ref[...] + agg @ wn_ref[...], 0.0)
    out96 = h @ wp_ref[...] + b_ref[...]
    sc = sc_ref[...]
    scb = sc @ r_ref[...]
    inv = (1.0 / sc) @ r_ref[...]
    m = msk_ref[...] > 0.0
    wt_ref[...] = out96 * scb * valid_ref[...]
    base_ref[...] = jnp.where(m, 0.0, out96)
    w_ref[...] = jnp.where(m, inv, 0.0)


def _kb_body(kbt_hbm, wt_hbm, base_hbm, w_hbm, out_hbm,
             kb_v, acc_v, bb_v, ww_v, sem, sem2):
    cid = lax.axis_index("c")
    sid = lax.axis_index("s")
    wid = sid * NC + cid
    npw = NP // NW
    n0 = wid * npw

    def batch(j, _):
        nb = n0 + j * NB
        bidx = nb // NB
        pltpu.sync_copy(kbt_hbm.at[bidx], kb_v)
        cb = pltpu.async_copy(base_hbm.at[pl.ds(nb, NB)], bb_v, sem2)
        cw = pltpu.async_copy(w_hbm.at[pl.ds(nb, NB)], ww_v, sem2)
        pltpu.async_copy(wt_hbm.at[kb_v.at[0]], acc_v, sem).wait()

        def kbody(k, _):
            pltpu.async_copy(wt_hbm.at[kb_v.at[k]], acc_v, sem,
                             add=True).wait()
            return 0
        lax.fori_loop(1, K, kbody, 0)
        cb.wait()
        cw.wait()

        def comb(i, _):
            for c in range(F // L):
                s = pl.ds(c * L, L)
                acc_v[i, s] = bb_v[i, s] + acc_v[i, s] * ww_v[i, s]
            return 0
        lax.fori_loop(0, NB, comb, 0)
        pltpu.sync_copy(acc_v, out_hbm.at[pl.ds(nb, NB)])
        return 0
    lax.fori_loop(0, npw // NB, batch, 0)


def kernel(x, edge_index, keybom, scaler, key_aggregation_status,
           W_self, W_neigh, W_proj, b_proj):
    f32 = jnp.float32
    i32 = jnp.int32
    E = edge_index.shape[1]
    epw = -(-E // (NW * EB)) * EB
    EP = epw * NW

    xp = jnp.zeros((NP, DP), f32).at[:N, :D].set(x).at[:N, D].set(1.0)
    srcp = jnp.full((EP,), N, i32).at[:E].set(edge_index[0])
    dstp = jnp.full((EP,), N, i32).at[:E].set(edge_index[1])
    kb = jnp.where(keybom < 0, N, keybom)
    kbt3 = (jnp.full((K, NP), N, i32).at[:, :N].set(kb.T)
            .reshape(K, NP // NB, NB).transpose(1, 0, 2))
    scp = jnp.ones((NP, TP), f32).at[:N, :T].set(scaler)
    mskf = jnp.zeros((NP, 1), f32).at[:N].set(
        (key_aggregation_status > 0).astype(f32))
    validf = jnp.zeros((NP, 1), f32).at[:N, :].set(1.0)
    wp96 = jnp.zeros((H, F), f32).at[:, :T * Q].set(W_proj)
    b96 = jnp.zeros((1, F), f32).at[0, :T * Q].set(b_proj)
    rmat = (jnp.arange(F)[None, :] // Q == jnp.arange(TP)[:, None]).astype(f32)

    mesh = plsc.VectorSubcoreMesh(core_axis_name="c", subcore_axis_name="s",
                                  num_cores=NC, num_subcores=NS)

    edge_fn = pl.kernel(
        functools.partial(_edge_body, epw=epw),
        out_type=jax.ShapeDtypeStruct((NC, NP, DP), f32),
        mesh=mesh,
        compiler_params=pltpu.CompilerParams(use_tc_tiling_on_sc=False),
        scratch_types=[
            pltpu.VMEM((EB,), i32),
            pltpu.VMEM((EB,), i32),
            pltpu.VMEM((EB, DP), f32),
            pltpu.VMEM_SHARED((NP, DP), f32),
            pltpu.SemaphoreType.DMA,
        ],
    )
    agg2 = edge_fn(srcp, dstp, xp)

    grid = NP // BLK
    wt, base, w = pl.pallas_call(
        _dense_body,
        grid=(grid,),
        in_specs=[
            pl.BlockSpec((BLK, DP), lambda i: (i, 0)),
            pl.BlockSpec((NC, BLK, DP), lambda i: (0, i, 0)),
            pl.BlockSpec((BLK, TP), lambda i: (i, 0)),
            pl.BlockSpec((BLK, 1), lambda i: (i, 0)),
            pl.BlockSpec((BLK, 1), lambda i: (i, 0)),
            pl.BlockSpec((D, H), lambda i: (0, 0)),
            pl.BlockSpec((D, H), lambda i: (0, 0)),
            pl.BlockSpec((H, F), lambda i: (0, 0)),
            pl.BlockSpec((1, F), lambda i: (0, 0)),
            pl.BlockSpec((TP, F), lambda i: (0, 0)),
        ],
        out_specs=[
            pl.BlockSpec((BLK, F), lambda i: (i, 0)),
            pl.BlockSpec((BLK, F), lambda i: (i, 0)),
            pl.BlockSpec((BLK, F), lambda i: (i, 0)),
        ],
        out_shape=[
            jax.ShapeDtypeStruct((NP, F), f32),
            jax.ShapeDtypeStruct((NP, F), f32),
            jax.ShapeDtypeStruct((NP, F), f32),
        ],
    )(xp, agg2, scp, mskf, validf, W_self, W_neigh, wp96, b96, rmat)

    kb_fn = pl.kernel(
        _kb_body,
        out_type=jax.ShapeDtypeStruct((NP, F), f32),
        mesh=mesh,
        compiler_params=pltpu.CompilerParams(use_tc_tiling_on_sc=False),
        scratch_types=[
            pltpu.VMEM((K, NB), i32),
            pltpu.VMEM((NB, F), f32),
            pltpu.VMEM((NB, F), f32),
            pltpu.VMEM((NB, F), f32),
            pltpu.SemaphoreType.DMA,
            pltpu.SemaphoreType.DMA,
        ],
    )
    outp = kb_fn(kbt3, wt, base, w)

    return outp[:N, :T * Q].reshape(N, T, Q)


# speedup vs baseline: 4.2838x; 1.1660x over previous
import functools

import jax
import jax.numpy as jnp
from jax import lax
from jax.experimental import pallas as pl
from jax.experimental.pallas import tpu as pltpu
from jax.experimental.pallas import tpu_sc as plsc

N = 10000
D = 128
H = 64
T = 28
Q = 3
K = 50

NC = 2
NS = 16
L = 16
NW = NC * NS

NP = 10240
DP = 144
F = 96
TP = 32

EB = 128
NB = 80
BLK = 512


def _edge_body(src_hbm, dst_hbm, xp_hbm, agg_hbm,
               sidx_v, didx_v, rows_v, agg_sh, semi, semg, sems, *, epw):
    cid = lax.axis_index("c")
    sid = lax.axis_index("s")
    wid = sid * NC + cid
    nbat = epw // EB

    def zrow(i, _):
        for c in range(DP // L):
            rows_v[0, i, pl.ds(c * L, L)] = jnp.zeros((L,), jnp.float32)
        return 0
    lax.fori_loop(0, EB, zrow, 0)
    zrows = NP // NS
    for z in range(zrows // EB):
        pltpu.sync_copy(rows_v.at[0],
                        agg_sh.at[pl.ds(sid * zrows + z * EB, EB)])
    plsc.subcore_barrier()

    e0 = wid * epw
    pltpu.async_copy(src_hbm.at[pl.ds(e0, EB)], sidx_v.at[0], semi)
    pltpu.async_copy(dst_hbm.at[pl.ds(e0, EB)], didx_v.at[0], semi)

    def body(j, _):
        b = j % 2
        base = e0 + j * EB
        pltpu.make_async_copy(src_hbm.at[pl.ds(base, EB)],
                              sidx_v.at[b], semi).wait()
        pltpu.make_async_copy(dst_hbm.at[pl.ds(base, EB)],
                              didx_v.at[b], semi).wait()

        @pl.when(j + 1 < nbat)
        def _():
            pltpu.async_copy(src_hbm.at[pl.ds(base + EB, EB)],
                             sidx_v.at[1 - b], semi)
            pltpu.async_copy(dst_hbm.at[pl.ds(base + EB, EB)],
                             didx_v.at[1 - b], semi)

        @pl.when(j >= 2)
        def _():
            pltpu.make_async_copy(rows_v.at[b],
                                  agg_sh.at[pl.ds(0, EB)], sems).wait()
        pltpu.async_copy(xp_hbm.at[sidx_v.at[b]], rows_v.at[b], semg).wait()
        pltpu.async_copy(rows_v.at[b], agg_sh.at[didx_v.at[b]], sems,
                         add=True)
        return 0
    lax.fori_loop(0, nbat, body, 0)
    pltpu.make_async_copy(rows_v.at[0], agg_sh.at[pl.ds(0, EB)], sems).wait()
    pltpu.make_async_copy(rows_v.at[1], agg_sh.at[pl.ds(0, EB)], sems).wait()
    plsc.subcore_barrier()

    pltpu.sync_copy(agg_sh.at[pl.ds(sid * zrows, zrows)],
                    agg_hbm.at[cid, pl.ds(sid * zrows, zrows)])


def _dense_body(xp_ref, agg_ref, sc_ref, msk_ref, valid_ref,
                ws_ref, wn_ref, wp_ref, b_ref, r_ref,
                wt_ref, base_ref, w_ref):
    a = agg_ref[0] + agg_ref[1]
    deg = jnp.maximum(a[:, D:D + 1], 1.0)
    agg = a[:, :D] / deg
    xb = xp_ref[...][:, :D]
    h = jnp.maximum(xb @ ws_ref[...] + agg @ wn_ref[...], 0.0)
    out96 = h @ wp_ref[...] + b_ref[...]
    sc = sc_ref[...]
    scb = sc @ r_ref[...]
    inv = (1.0 / sc) @ r_ref[...]
    m = msk_ref[...] > 0.0
    wt_ref[...] = out96 * scb * valid_ref[...]
    base_ref[...] = jnp.where(m, 0.0, out96)
    w_ref[...] = jnp.where(m, inv, 0.0)


def _kb_body(kbt_hbm, wt_hbm, base_hbm, w_hbm, out_hbm,
             kb_v, acc_v, bb_v, ww_v, sem, sem2):
    cid = lax.axis_index("c")
    sid = lax.axis_index("s")
    wid = sid * NC + cid
    npw = NP // NW
    n0 = wid * npw

    def batch(j, _):
        nb = n0 + j * NB
        bidx = nb // NB
        pltpu.sync_copy(kbt_hbm.at[bidx], kb_v)
        cb = pltpu.async_copy(base_hbm.at[pl.ds(nb, NB)], bb_v, sem2)
        cw = pltpu.async_copy(w_hbm.at[pl.ds(nb, NB)], ww_v, sem2)
        pltpu.async_copy(wt_hbm.at[kb_v.at[0]], acc_v, sem).wait()

        W = 8

        def kfire(k, _):
            pltpu.async_copy(wt_hbm.at[kb_v.at[k]], acc_v, sem, add=True)

            @pl.when(k >= W + 1)
            def _():
                pltpu.make_async_copy(wt_hbm.at[kb_v.at[0]], acc_v,
                                      sem).wait()
            return 0
        lax.fori_loop(1, K, kfire, 0)

        def kdrain(k, _):
            pltpu.make_async_copy(wt_hbm.at[kb_v.at[0]], acc_v, sem).wait()
            return 0
        lax.fori_loop(0, W, kdrain, 0)
        cb.wait()
        cw.wait()

        def comb(i, _):
            for c in range(F // L):
                s = pl.ds(c * L, L)
                acc_v[i, s] = bb_v[i, s] + acc_v[i, s] * ww_v[i, s]
            return 0
        lax.fori_loop(0, NB, comb, 0)
        pltpu.sync_copy(acc_v, out_hbm.at[pl.ds(nb, NB)])
        return 0
    lax.fori_loop(0, npw // NB, batch, 0)


def kernel(x, edge_index, keybom, scaler, key_aggregation_status,
           W_self, W_neigh, W_proj, b_proj):
    f32 = jnp.float32
    i32 = jnp.int32
    E = edge_index.shape[1]
    epw = -(-E // (NW * EB)) * EB
    EP = epw * NW

    xp = jnp.zeros((NP, DP), f32).at[:N, :D].set(x).at[:N, D].set(1.0)
    srcp = jnp.full((EP,), N, i32).at[:E].set(edge_index[0])
    dstp = jnp.full((EP,), N, i32).at[:E].set(edge_index[1])
    kb = jnp.where(keybom < 0, N, keybom)
    kbt3 = (jnp.full((K, NP), N, i32).at[:, :N].set(kb.T)
            .reshape(K, NP // NB, NB).transpose(1, 0, 2))
    scp = jnp.ones((NP, TP), f32).at[:N, :T].set(scaler)
    mskf = jnp.zeros((NP, 1), f32).at[:N].set(
        (key_aggregation_status > 0).astype(f32))
    validf = jnp.zeros((NP, 1), f32).at[:N, :].set(1.0)
    wp96 = jnp.zeros((H, F), f32).at[:, :T * Q].set(W_proj)
    b96 = jnp.zeros((1, F), f32).at[0, :T * Q].set(b_proj)
    rmat = (jnp.arange(F)[None, :] // Q == jnp.arange(TP)[:, None]).astype(f32)

    mesh = plsc.VectorSubcoreMesh(core_axis_name="c", subcore_axis_name="s",
                                  num_cores=NC, num_subcores=NS)

    edge_fn = pl.kernel(
        functools.partial(_edge_body, epw=epw),
        out_type=jax.ShapeDtypeStruct((NC, NP, DP), f32),
        mesh=mesh,
        compiler_params=pltpu.CompilerParams(use_tc_tiling_on_sc=False),
        scratch_types=[
            pltpu.VMEM((2, EB), i32),
            pltpu.VMEM((2, EB), i32),
            pltpu.VMEM((2, EB, DP), f32),
            pltpu.VMEM_SHARED((NP, DP), f32),
            pltpu.SemaphoreType.DMA,
            pltpu.SemaphoreType.DMA,
            pltpu.SemaphoreType.DMA,
        ],
    )
    agg2 = edge_fn(srcp, dstp, xp)

    grid = NP // BLK
    wt, base, w = pl.pallas_call(
        _dense_body,
        grid=(grid,),
        in_specs=[
            pl.BlockSpec((BLK, DP), lambda i: (i, 0)),
            pl.BlockSpec((NC, BLK, DP), lambda i: (0, i, 0)),
            pl.BlockSpec((BLK, TP), lambda i: (i, 0)),
            pl.BlockSpec((BLK, 1), lambda i: (i, 0)),
            pl.BlockSpec((BLK, 1), lambda i: (i, 0)),
            pl.BlockSpec((D, H), lambda i: (0, 0)),
            pl.BlockSpec((D, H), lambda i: (0, 0)),
            pl.BlockSpec((H, F), lambda i: (0, 0)),
            pl.BlockSpec((1, F), lambda i: (0, 0)),
            pl.BlockSpec((TP, F), lambda i: (0, 0)),
        ],
        out_specs=[
            pl.BlockSpec((BLK, F), lambda i: (i, 0)),
            pl.BlockSpec((BLK, F), lambda i: (i, 0)),
            pl.BlockSpec((BLK, F), lambda i: (i, 0)),
        ],
        out_shape=[
            jax.ShapeDtypeStruct((NP, F), f32),
            jax.ShapeDtypeStruct((NP, F), f32),
            jax.ShapeDtypeStruct((NP, F), f32),
        ],
    )(xp, agg2, scp, mskf, validf, W_self, W_neigh, wp96, b96, rmat)

    kb_fn = pl.kernel(
        _kb_body,
        out_type=jax.ShapeDtypeStruct((NP, F), f32),
        mesh=mesh,
        compiler_params=pltpu.CompilerParams(use_tc_tiling_on_sc=False),
        scratch_types=[
            pltpu.VMEM((K, NB), i32),
            pltpu.VMEM((NB, F), f32),
            pltpu.VMEM((NB, F), f32),
            pltpu.VMEM((NB, F), f32),
            pltpu.SemaphoreType.DMA,
            pltpu.SemaphoreType.DMA,
        ],
    )
    outp = kb_fn(kbt3, wt, base, w)

    return outp[:N, :T * Q].reshape(N, T, Q)
